# Initial kernel scaffold; baseline (speedup 1.0000x reference)
import functools

import jax
import jax.numpy as jnp
from jax import lax
from jax.experimental import pallas as pl
from jax.experimental.pallas import tpu as pltpu
from jax.experimental.pallas import tpu_sc as plsc

N = 10000
E = 320000
D = 128
G = 64

NC = 2
NS = 16
NW = NC * NS

E_PER_CORE = E // NC
E_PER_TILE = E // NW
CHUNK = 128
N_FULL = E_PER_TILE // CHUNK
REM = E_PER_TILE - N_FULL * CHUNK
ROWS_PER_TILE = N // NS


def _zero_vmem_2d(ref, nrows):
    zero = jnp.zeros((16,), jnp.float32)
    ncols = ref.shape[1]

    def body(i, c):
        for u in range(ncols // 16):
            ref[i, pl.ds(u * 16, 16)] = zero
        return c

    lax.fori_loop(0, nrows, body, 0)


def _fill_ones_vmem_2d(ref, nrows):
    one = jnp.ones((16,), jnp.float32)
    ncols = ref.shape[1]

    def body(i, c):
        for u in range(ncols // 16):
            ref[i, pl.ds(u * 16, 16)] = one
        return c

    lax.fori_loop(0, nrows, body, 0)


def _zero_spmem_slice(acc_sh, row0, nrows, zbuf, zrows):
    nfull = nrows // zrows
    rem = nrows - nfull * zrows
    for k in range(nfull):
        pltpu.sync_copy(zbuf, acc_sh.at[pl.ds(row0 + k * zrows, zrows)])
    if rem:
        pltpu.sync_copy(zbuf.at[pl.ds(0, rem)],
                        acc_sh.at[pl.ds(row0 + nfull * zrows, rem)])


def _sc_deg_body(dst_hbm, out_hbm, ones_v, zbuf, idx_v, idx16_v, acc_sh):
    cid = lax.axis_index("c")
    sid = lax.axis_index("s")

    _zero_vmem_2d(zbuf, 128)
    _zero_spmem_slice(acc_sh, sid * ROWS_PER_TILE, ROWS_PER_TILE, zbuf, 128)
    _fill_ones_vmem_2d(ones_v, CHUNK)
    plsc.subcore_barrier()

    ebase = cid * E_PER_CORE + sid * E_PER_TILE

    def body(j, c):
        base = ebase + j * CHUNK
        pltpu.sync_copy(dst_hbm.at[pl.ds(base, CHUNK)], idx_v)
        pltpu.sync_copy(ones_v, acc_sh.at[idx_v], add=True)
        return c

    lax.fori_loop(0, N_FULL, body, 0)

    rbase = ebase + N_FULL * CHUNK
    pltpu.sync_copy(dst_hbm.at[pl.ds(rbase, REM)], idx16_v)
    pltpu.sync_copy(ones_v.at[pl.ds(0, REM)], acc_sh.at[idx16_v], add=True)

    plsc.subcore_barrier()
    row0 = sid * ROWS_PER_TILE
    pltpu.sync_copy(acc_sh.at[pl.ds(row0, ROWS_PER_TILE)],
                    out_hbm.at[cid, pl.ds(row0, ROWS_PER_TILE)])


def _sc_deg(dst):
    mesh = plsc.VectorSubcoreMesh(core_axis_name="c", subcore_axis_name="s")
    k = functools.partial(
        pl.kernel,
        out_type=jax.ShapeDtypeStruct((NC, N, 16), jnp.float32),
        mesh=mesh,
        scratch_types=[
            pltpu.VMEM((CHUNK, 16), jnp.float32),
            pltpu.VMEM((128, 16), jnp.float32),
            pltpu.VMEM((CHUNK,), jnp.int32),
            pltpu.VMEM((REM,), jnp.int32),
            pltpu.VMEM_SHARED((N, 16), jnp.float32),
        ],
    )(_sc_deg_body)
    return k(dst)


def _sc_agg_body(ht_hbm, src_hbm, dst_hbm, out_hbm,
                 sidx_v, didx_v, sidx16_v, didx16_v,
                 rows_v, rows16_v, acc_sh, sem):
    cid = lax.axis_index("c")
    sid = lax.axis_index("s")

    _zero_vmem_2d(rows_v, 128)
    _zero_spmem_slice(acc_sh, sid * ROWS_PER_TILE, ROWS_PER_TILE, rows_v, 128)
    plsc.subcore_barrier()

    ebase = cid * E_PER_CORE + sid * E_PER_TILE

    def body(j, c):
        base = ebase + j * CHUNK
        pltpu.sync_copy(src_hbm.at[pl.ds(base, CHUNK)], sidx_v)
        pltpu.sync_copy(dst_hbm.at[pl.ds(base, CHUNK)], didx_v)
        pltpu.async_copy(ht_hbm.at[sidx_v], rows_v, sem).wait()
        pltpu.sync_copy(rows_v, acc_sh.at[didx_v], add=True)
        return c

    lax.fori_loop(0, N_FULL, body, 0)

    rbase = ebase + N_FULL * CHUNK
    pltpu.sync_copy(src_hbm.at[pl.ds(rbase, REM)], sidx16_v)
    pltpu.sync_copy(dst_hbm.at[pl.ds(rbase, REM)], didx16_v)
    pltpu.async_copy(ht_hbm.at[sidx16_v], rows16_v, sem).wait()
    pltpu.sync_copy(rows16_v, acc_sh.at[didx16_v], add=True)

    plsc.subcore_barrier()
    row0 = sid * ROWS_PER_TILE
    pltpu.sync_copy(acc_sh.at[pl.ds(row0, ROWS_PER_TILE)],
                    out_hbm.at[cid, pl.ds(row0, ROWS_PER_TILE)])


def _sc_agg(ht, src, dst):
    mesh = plsc.VectorSubcoreMesh(core_axis_name="c", subcore_axis_name="s")
    k = functools.partial(
        pl.kernel,
        out_type=jax.ShapeDtypeStruct((NC, N, D), jnp.float32),
        mesh=mesh,
        scratch_types=[
            pltpu.VMEM((CHUNK,), jnp.int32),
            pltpu.VMEM((CHUNK,), jnp.int32),
            pltpu.VMEM((REM,), jnp.int32),
            pltpu.VMEM((REM,), jnp.int32),
            pltpu.VMEM((CHUNK, D), jnp.float32),
            pltpu.VMEM((REM, D), jnp.float32),
            pltpu.VMEM_SHARED((N, D), jnp.float32),
            pltpu.SemaphoreType.DMA,
        ],
    )(_sc_agg_body)
    return k(ht, src, dst)


_HI = jax.lax.Precision.HIGHEST


def _tc1_body(x_ref, w_ref, degp_ref, dis_ref, ht_ref):
    degp = degp_ref[...]
    deg = degp[0, :, 0:1] + degp[1, :, 0:1] + 1.0
    dis = lax.rsqrt(deg)
    h = jnp.dot(x_ref[...], w_ref[...],
                preferred_element_type=jnp.float32, precision=_HI)
    dis_ref[...] = dis
    ht_ref[...] = h * dis


def _tc1(x, W1, degp):
    return pl.pallas_call(
        _tc1_body,
        out_shape=[
            jax.ShapeDtypeStruct((N, 1), jnp.float32),
            jax.ShapeDtypeStruct((N, D), jnp.float32),
        ],
    )(x, W1, degp)


def _tc_mid_body(accp_ref, ht_ref, dis_ref, b_ref, w_ref, out_ref):
    agg = accp_ref[0] + accp_ref[1] + ht_ref[...]
    z = jnp.maximum(dis_ref[...] * agg + b_ref[...], 0.0)
    h = jnp.dot(z, w_ref[...],
                preferred_element_type=jnp.float32, precision=_HI)
    out_ref[...] = h * dis_ref[...]


def _tc_mid(accp, ht, dis, b, W):
    return pl.pallas_call(
        _tc_mid_body,
        out_shape=jax.ShapeDtypeStruct((N, D), jnp.float32),
    )(accp, ht, dis, b.reshape(1, D), W)


def _tc_final_body(accp_ref, ht_ref, dis_ref, b_ref, batch_ref, out_ref):
    agg = accp_ref[0] + accp_ref[1] + ht_ref[...]
    z = jnp.maximum(dis_ref[...] * agg + b_ref[...], 0.0)
    seg = lax.broadcasted_iota(jnp.int32, (N, G), 1)
    onehot = (seg == batch_ref[...]).astype(jnp.float32)
    pool = lax.dot_general(onehot, z, (((0,), (0,)), ((), ())),
                           preferred_element_type=jnp.float32,
                           precision=_HI)
    out_ref[...] = pool


def _tc_final(accp, ht, dis, b, batch):
    return pl.pallas_call(
        _tc_final_body,
        out_shape=jax.ShapeDtypeStruct((G, D), jnp.float32),
    )(accp, ht, dis, b.reshape(1, D), batch.reshape(N, 1))


def kernel(x, edge_index, batch, W1, b1, W2, b2):
    src = edge_index[0]
    dst = edge_index[1]

    degp = _sc_deg(dst)
    dis, ht1 = _tc1(x, W1, degp)
    acc1 = _sc_agg(ht1, src, dst)
    ht2 = _tc_mid(acc1, ht1, dis, b1, W2)
    acc2 = _sc_agg(ht2, src, dst)
    return _tc_final(acc2, ht2, dis, b2, batch)


# speedup vs baseline: 16.3476x; 16.3476x over previous
import functools

import jax
import jax.numpy as jnp
from jax import lax
from jax.experimental import pallas as pl
from jax.experimental.pallas import tpu as pltpu
from jax.experimental.pallas import tpu_sc as plsc

N = 10000
E = 320000
D = 128
G = 64

NC = 2
NS = 16
NW = NC * NS

E_PER_CORE = E // NC
E_PER_TILE = E // NW
CHUNK = 128
N_FULL = E_PER_TILE // CHUNK
REM = E_PER_TILE - N_FULL * CHUNK
ROW_STEP = 624
ROW_SPAN = 640


def _zero_vmem_2d(ref, nrows):
    zero = jnp.zeros((16,), jnp.float32)
    ncols = ref.shape[1]

    def body(i, c):
        for u in range(ncols // 16):
            ref[i, pl.ds(u * 16, 16)] = zero
        return c

    lax.fori_loop(0, nrows, body, 0)


def _fill_ones_vmem_2d(ref, nrows):
    one = jnp.ones((16,), jnp.float32)
    ncols = ref.shape[1]

    def body(i, c):
        for u in range(ncols // 16):
            ref[i, pl.ds(u * 16, 16)] = one
        return c

    lax.fori_loop(0, nrows, body, 0)


def _zero_spmem_slice(acc_sh, row0, nrows, zbuf, zrows):
    nfull = nrows // zrows
    rem = nrows - nfull * zrows
    for k in range(nfull):
        pltpu.sync_copy(zbuf, acc_sh.at[pl.ds(row0 + k * zrows, zrows)])
    if rem:
        pltpu.sync_copy(zbuf.at[pl.ds(0, rem)],
                        acc_sh.at[pl.ds(row0 + nfull * zrows, rem)])


def _sc_deg_body(dst_hbm, out_hbm, ones_v, zbuf, idx_v, idx16_v, acc_sh):
    cid = lax.axis_index("c")
    sid = lax.axis_index("s")

    _zero_vmem_2d(zbuf, 128)
    _zero_spmem_slice(acc_sh, sid * ROW_STEP, ROW_SPAN, zbuf, 128)
    _fill_ones_vmem_2d(ones_v, CHUNK)
    plsc.subcore_barrier()

    ebase = cid * E_PER_CORE + sid * E_PER_TILE

    def body(j, c):
        base = ebase + j * CHUNK
        pltpu.sync_copy(dst_hbm.at[pl.ds(base, CHUNK)], idx_v)
        pltpu.sync_copy(ones_v, acc_sh.at[idx_v], add=True)
        return c

    lax.fori_loop(0, N_FULL, body, 0)

    rbase = ebase + N_FULL * CHUNK
    pltpu.sync_copy(dst_hbm.at[pl.ds(rbase, REM)], idx16_v)
    pltpu.sync_copy(ones_v.at[pl.ds(0, REM)], acc_sh.at[idx16_v], add=True)

    plsc.subcore_barrier()
    row0 = sid * ROW_STEP
    pltpu.sync_copy(acc_sh.at[pl.ds(row0, ROW_SPAN)],
                    out_hbm.at[cid, pl.ds(row0, ROW_SPAN)])


def _sc_deg(dst):
    mesh = plsc.VectorSubcoreMesh(core_axis_name="c", subcore_axis_name="s")
    k = functools.partial(
        pl.kernel,
        out_type=jax.ShapeDtypeStruct((NC, N, 16), jnp.float32),
        mesh=mesh,
        scratch_types=[
            pltpu.VMEM((CHUNK, 16), jnp.float32),
            pltpu.VMEM((128, 16), jnp.float32),
            pltpu.VMEM((CHUNK,), jnp.int32),
            pltpu.VMEM((REM,), jnp.int32),
            pltpu.VMEM_SHARED((N, 16), jnp.float32),
        ],
    )(_sc_deg_body)
    return k(dst)


def _sc_agg_body(ht_hbm, src_hbm, dst_hbm, out_hbm,
                 sidx_v, didx_v, sidx16_v, didx16_v,
                 rows_v, rows16_v, acc_sh, sem):
    cid = lax.axis_index("c")
    sid = lax.axis_index("s")

    _zero_vmem_2d(rows_v, 128)
    _zero_spmem_slice(acc_sh, sid * ROW_STEP, ROW_SPAN, rows_v, 128)
    plsc.subcore_barrier()

    ebase = cid * E_PER_CORE + sid * E_PER_TILE

    def body(j, c):
        base = ebase + j * CHUNK
        pltpu.sync_copy(src_hbm.at[pl.ds(base, CHUNK)], sidx_v)
        pltpu.sync_copy(dst_hbm.at[pl.ds(base, CHUNK)], didx_v)
        pltpu.async_copy(ht_hbm.at[sidx_v], rows_v, sem).wait()
        pltpu.sync_copy(rows_v, acc_sh.at[didx_v], add=True)
        return c

    lax.fori_loop(0, N_FULL, body, 0)

    rbase = ebase + N_FULL * CHUNK
    pltpu.sync_copy(src_hbm.at[pl.ds(rbase, REM)], sidx16_v)
    pltpu.sync_copy(dst_hbm.at[pl.ds(rbase, REM)], didx16_v)
    pltpu.async_copy(ht_hbm.at[sidx16_v], rows16_v, sem).wait()
    pltpu.sync_copy(rows16_v, acc_sh.at[didx16_v], add=True)

    plsc.subcore_barrier()
    row0 = sid * ROW_STEP
    pltpu.sync_copy(acc_sh.at[pl.ds(row0, ROW_SPAN)],
                    out_hbm.at[cid, pl.ds(row0, ROW_SPAN)])


def _sc_agg(ht, src, dst):
    mesh = plsc.VectorSubcoreMesh(core_axis_name="c", subcore_axis_name="s")
    k = functools.partial(
        pl.kernel,
        out_type=jax.ShapeDtypeStruct((NC, N, D), jnp.float32),
        mesh=mesh,
        scratch_types=[
            pltpu.VMEM((CHUNK,), jnp.int32),
            pltpu.VMEM((CHUNK,), jnp.int32),
            pltpu.VMEM((REM,), jnp.int32),
            pltpu.VMEM((REM,), jnp.int32),
            pltpu.VMEM((CHUNK, D), jnp.float32),
            pltpu.VMEM((REM, D), jnp.float32),
            pltpu.VMEM_SHARED((N, D), jnp.float32),
            pltpu.SemaphoreType.DMA,
        ],
    )(_sc_agg_body)
    return k(ht, src, dst)


_HI = jax.lax.Precision.HIGHEST


def _tc1_body(x_ref, w_ref, degp_ref, dis_ref, ht_ref):
    degp = degp_ref[...]
    deg = degp[0, :, 0:1] + degp[1, :, 0:1] + 1.0
    dis = lax.rsqrt(deg)
    h = jnp.dot(x_ref[...], w_ref[...],
                preferred_element_type=jnp.float32, precision=_HI)
    dis_ref[...] = dis
    ht_ref[...] = h * dis


def _tc1(x, W1, degp):
    return pl.pallas_call(
        _tc1_body,
        out_shape=[
            jax.ShapeDtypeStruct((N, 1), jnp.float32),
            jax.ShapeDtypeStruct((N, D), jnp.float32),
        ],
    )(x, W1, degp)


def _tc_mid_body(accp_ref, ht_ref, dis_ref, b_ref, w_ref, out_ref):
    agg = accp_ref[0] + accp_ref[1] + ht_ref[...]
    z = jnp.maximum(dis_ref[...] * agg + b_ref[...], 0.0)
    h = jnp.dot(z, w_ref[...],
                preferred_element_type=jnp.float32, precision=_HI)
    out_ref[...] = h * dis_ref[...]


def _tc_mid(accp, ht, dis, b, W):
    return pl.pallas_call(
        _tc_mid_body,
        out_shape=jax.ShapeDtypeStruct((N, D), jnp.float32),
    )(accp, ht, dis, b.reshape(1, D), W)


def _tc_final_body(accp_ref, ht_ref, dis_ref, b_ref, batch_ref, out_ref):
    agg = accp_ref[0] + accp_ref[1] + ht_ref[...]
    z = jnp.maximum(dis_ref[...] * agg + b_ref[...], 0.0)
    seg = lax.broadcasted_iota(jnp.int32, (N, G), 1)
    onehot = (seg == batch_ref[...]).astype(jnp.float32)
    pool = lax.dot_general(onehot, z, (((0,), (0,)), ((), ())),
                           preferred_element_type=jnp.float32,
                           precision=_HI)
    out_ref[...] = pool


def _tc_final(accp, ht, dis, b, batch):
    return pl.pallas_call(
        _tc_final_body,
        out_shape=jax.ShapeDtypeStruct((G, D), jnp.float32),
    )(accp, ht, dis, b.reshape(1, D), batch.reshape(N, 1))


def kernel(x, edge_index, batch, W1, b1, W2, b2):
    src = edge_index[0]
    dst = edge_index[1]

    degp = _sc_deg(dst)
    dis, ht1 = _tc1(x, W1, degp)
    acc1 = _sc_agg(ht1, src, dst)
    ht2 = _tc_mid(acc1, ht1, dis, b1, W2)
    acc2 = _sc_agg(ht2, src, dst)
    return _tc_final(acc2, ht2, dis, b2, batch)
